# baseline (speedup 1.0000x reference)
import functools

import jax
import jax.numpy as jnp
from jax import lax
from jax.experimental import pallas as pl
from jax.experimental.pallas import tpu as pltpu
from jax.experimental.pallas import tpu_sc as plsc


def _argmin_body(z_ref, x2_ref, cb_ref, lat_ref, *, kc, nslot):
    zb = z_ref[0]
    zs = zb + zb
    tq = zs.shape[1]
    qbase = pl.program_id(0) * tq
    x2 = x2_ref[pl.ds(qbase, tq)].reshape(1, tq)
    k_total = cb_ref.shape[0]
    nrow = kc // nslot
    subf = lax.broadcasted_iota(jnp.int32, (nslot, tq), 0).astype(jnp.float32)
    acc_val = jnp.full((nslot, tq), jnp.inf, jnp.float32)
    acc_kf = jnp.zeros((nslot, tq), jnp.float32)
    for c in range(k_total // kc):
        cbc = cb_ref[pl.ds(c * kc, kc), :]
        c2c = jnp.sum(cbc * cbc, axis=1, keepdims=True)
        dots2 = lax.dot_general(
            cbc, zs, (((1,), (0,)), ((), ())),
            preferred_element_type=jnp.float32)
        dist = (x2 - dots2) + c2c
        for j in range(nrow):
            dv = dist[j * nslot:(j + 1) * nslot, :]
            kf = subf + jnp.float32(c * kc + j * nslot)
            b = dv < acc_val
            acc_val = jnp.where(b, dv, acc_val)
            acc_kf = jnp.where(b, kf, acc_kf)
    m = jnp.min(acc_val, axis=0, keepdims=True)
    cand = jnp.where(acc_val == m, acc_kf, jnp.float32(2.0**24))
    idxf = jnp.min(cand, axis=0, keepdims=True)
    lat_ref[...] = idxf.astype(jnp.int32).reshape(tq)


def _tc_argmin(z, x2, cb, *, tq=1024, kc=1024, nslot=32, interpret=False):
    b, d, t = z.shape
    nq = b * t
    k = cb.shape[0]
    tpb = t // tq
    body = functools.partial(_argmin_body, kc=kc, nslot=nslot)
    return pl.pallas_call(
        body,
        grid=(nq // tq,),
        in_specs=[
            pl.BlockSpec((1, d, tq), lambda i: (i // tpb, 0, i % tpb)),
            pl.BlockSpec((nq,), lambda i: (0,)),
            pl.BlockSpec((k, d), lambda i: (0, 0)),
        ],
        out_specs=pl.BlockSpec((tq,), lambda i: (i,)),
        out_shape=jax.ShapeDtypeStruct((nq,), jnp.int32),
        interpret=interpret,
    )(z, x2, cb)


def _sc_gather(cb, lat):
    k, d = cb.shape
    nq = lat.shape[0]
    info = plsc.get_sparse_core_info()
    nw = info.num_cores * info.num_subcores
    b_per_w = nq // nw
    chunk = min(b_per_w, 128)
    n_chunks = b_per_w // chunk
    mesh = plsc.VectorSubcoreMesh(core_axis_name="c", subcore_axis_name="s")

    @functools.partial(
        pl.kernel,
        mesh=mesh,
        compiler_params=pltpu.CompilerParams(use_tc_tiling_on_sc=False),
        out_type=jax.ShapeDtypeStruct((nq, d), jnp.float32),
        scratch_types=[
            pltpu.VMEM((b_per_w,), jnp.int32),
            pltpu.VMEM((chunk, d), jnp.float32),
            pltpu.SemaphoreType.DMA,
        ],
    )
    def gather_kernel(cb_hbm, idx_hbm, out_hbm, idx_v, rows_v, sem):
        wid = lax.axis_index("s") * info.num_cores + lax.axis_index("c")
        base = wid * b_per_w
        pltpu.sync_copy(idx_hbm.at[pl.ds(base, b_per_w)], idx_v)
        for j in range(n_chunks):
            pltpu.async_copy(cb_hbm.at[idx_v.at[pl.ds(j * chunk, chunk)]],
                             rows_v, sem).wait()
            pltpu.sync_copy(rows_v, out_hbm.at[pl.ds(base + j * chunk, chunk)])

    return gather_kernel(cb, lat)


def _finish_body(z_ref, q_ref, qst_ref, qt_ref):
    zb = z_ref[0]
    qb = q_ref[...]
    qt = qb.T
    qst_ref[0] = zb + (qt - zb)
    qt_ref[0] = qt


def _tc_finish(z, q, *, interpret=False):
    b, d, t = z.shape
    out = jax.ShapeDtypeStruct((b, d, t), jnp.float32)
    return pl.pallas_call(
        _finish_body,
        grid=(b,),
        in_specs=[
            pl.BlockSpec((1, d, t), lambda i: (i, 0, 0)),
            pl.BlockSpec((t, d), lambda i: (i, 0)),
        ],
        out_specs=[
            pl.BlockSpec((1, d, t), lambda i: (i, 0, 0)),
            pl.BlockSpec((1, d, t), lambda i: (i, 0, 0)),
        ],
        out_shape=[out, out],
        interpret=interpret,
    )(z, q)


def kernel(z, codebook):
    b, d, t = z.shape
    nq = b * t
    x_btd = jnp.transpose(z, (0, 2, 1))
    x2 = jnp.sum(x_btd * x_btd, axis=-1, keepdims=True).reshape(nq)
    lat = _tc_argmin(z, x2, codebook)
    q = _sc_gather(codebook, lat)
    qst_t, q_t = _tc_finish(z, q)
    return (qst_t, q_t)


# speedup vs baseline: 1.0602x; 1.0602x over previous
import functools

import jax
import jax.numpy as jnp
from jax import lax
from jax.experimental import pallas as pl
from jax.experimental.pallas import tpu as pltpu
from jax.experimental.pallas import tpu_sc as plsc


def _argmin_body(z_ref, x2_ref, cb_ref, lat_ref, *, kc, nslot):
    zb = z_ref[0]
    zs = zb + zb
    tq = zs.shape[1]
    qbase = pl.program_id(0) * tq
    x2 = x2_ref[pl.ds(qbase, tq)].reshape(1, tq)
    k_total = cb_ref.shape[0]
    nrow = kc // nslot
    subf = lax.broadcasted_iota(jnp.int32, (nslot, tq), 0).astype(jnp.float32)
    acc_val = jnp.full((nslot, tq), jnp.inf, jnp.float32)
    acc_kf = jnp.zeros((nslot, tq), jnp.float32)
    for c in range(k_total // kc):
        cbc = cb_ref[pl.ds(c * kc, kc), :]
        c2c = jnp.sum(cbc * cbc, axis=1, keepdims=True)
        dots2 = lax.dot_general(
            cbc, zs, (((1,), (0,)), ((), ())),
            preferred_element_type=jnp.float32)
        dist = (x2 - dots2) + c2c
        for j in range(nrow):
            dv = dist[j * nslot:(j + 1) * nslot, :]
            kf = subf + jnp.float32(c * kc + j * nslot)
            b = dv < acc_val
            acc_val = jnp.where(b, dv, acc_val)
            acc_kf = jnp.where(b, kf, acc_kf)
    m = jnp.min(acc_val, axis=0, keepdims=True)
    cand = jnp.where(acc_val == m, acc_kf, jnp.float32(2.0**24))
    idxf = jnp.min(cand, axis=0, keepdims=True)
    lat_ref[...] = idxf.astype(jnp.int32).reshape(tq)


def _tc_argmin(z, x2, cb, *, tq=2048, kc=2048, nslot=8, interpret=False):
    b, d, t = z.shape
    nq = b * t
    k = cb.shape[0]
    tpb = t // tq
    body = functools.partial(_argmin_body, kc=kc, nslot=nslot)
    return pl.pallas_call(
        body,
        grid=(nq // tq,),
        in_specs=[
            pl.BlockSpec((1, d, tq), lambda i: (i // tpb, 0, i % tpb)),
            pl.BlockSpec((nq,), lambda i: (0,)),
            pl.BlockSpec((k, d), lambda i: (0, 0)),
        ],
        out_specs=pl.BlockSpec((tq,), lambda i: (i,)),
        out_shape=jax.ShapeDtypeStruct((nq,), jnp.int32),
        interpret=interpret,
    )(z, x2, cb)


def _sc_gather(cb, lat):
    k, d = cb.shape
    nq = lat.shape[0]
    info = plsc.get_sparse_core_info()
    nw = info.num_cores * info.num_subcores
    b_per_w = nq // nw
    chunk = min(b_per_w, 128)
    n_chunks = b_per_w // chunk
    mesh = plsc.VectorSubcoreMesh(core_axis_name="c", subcore_axis_name="s")

    @functools.partial(
        pl.kernel,
        mesh=mesh,
        compiler_params=pltpu.CompilerParams(use_tc_tiling_on_sc=False),
        out_type=jax.ShapeDtypeStruct((nq, d), jnp.float32),
        scratch_types=[
            pltpu.VMEM((b_per_w,), jnp.int32),
            pltpu.VMEM((chunk, d), jnp.float32),
            pltpu.SemaphoreType.DMA,
        ],
    )
    def gather_kernel(cb_hbm, idx_hbm, out_hbm, idx_v, rows_v, sem):
        wid = lax.axis_index("s") * info.num_cores + lax.axis_index("c")
        base = wid * b_per_w
        pltpu.sync_copy(idx_hbm.at[pl.ds(base, b_per_w)], idx_v)
        for j in range(n_chunks):
            pltpu.async_copy(cb_hbm.at[idx_v.at[pl.ds(j * chunk, chunk)]],
                             rows_v, sem).wait()
            pltpu.sync_copy(rows_v, out_hbm.at[pl.ds(base + j * chunk, chunk)])

    return gather_kernel(cb, lat)


def _finish_body(z_ref, q_ref, qst_ref, qt_ref):
    zb = z_ref[0]
    qb = q_ref[...]
    qt = qb.T
    qst_ref[0] = zb + (qt - zb)
    qt_ref[0] = qt


def _tc_finish(z, q, *, interpret=False):
    b, d, t = z.shape
    out = jax.ShapeDtypeStruct((b, d, t), jnp.float32)
    return pl.pallas_call(
        _finish_body,
        grid=(b,),
        in_specs=[
            pl.BlockSpec((1, d, t), lambda i: (i, 0, 0)),
            pl.BlockSpec((t, d), lambda i: (i, 0)),
        ],
        out_specs=[
            pl.BlockSpec((1, d, t), lambda i: (i, 0, 0)),
            pl.BlockSpec((1, d, t), lambda i: (i, 0, 0)),
        ],
        out_shape=[out, out],
        interpret=interpret,
    )(z, q)


def kernel(z, codebook):
    b, d, t = z.shape
    nq = b * t
    x_btd = jnp.transpose(z, (0, 2, 1))
    x2 = jnp.sum(x_btd * x_btd, axis=-1, keepdims=True).reshape(nq)
    lat = _tc_argmin(z, x2, codebook)
    q = _sc_gather(codebook, lat)
    qst_t, q_t = _tc_finish(z, q)
    return (qst_t, q_t)


# speedup vs baseline: 1.0839x; 1.0224x over previous
import functools

import jax
import jax.numpy as jnp
from jax import lax
from jax.experimental import pallas as pl
from jax.experimental.pallas import tpu as pltpu
from jax.experimental.pallas import tpu_sc as plsc


def _argmin_body(z_ref, cb_ref, lat_ref, *, kc, nslot):
    zb = z_ref[0]
    zs = zb + zb
    tq = zs.shape[1]
    x2 = jnp.sum(zb * zb, axis=0, keepdims=True)
    k_total = cb_ref.shape[0]
    nrow = kc // nslot
    subf = lax.broadcasted_iota(jnp.int32, (nslot, tq), 0).astype(jnp.float32)
    acc_val = jnp.full((nslot, tq), jnp.inf, jnp.float32)
    acc_kf = jnp.zeros((nslot, tq), jnp.float32)
    for c in range(k_total // kc):
        cbc = cb_ref[pl.ds(c * kc, kc), :]
        c2c = jnp.sum(cbc * cbc, axis=1, keepdims=True)
        dots2 = lax.dot_general(
            cbc, zs, (((1,), (0,)), ((), ())),
            preferred_element_type=jnp.float32)
        dist = (x2 - dots2) + c2c
        for j in range(nrow):
            dv = dist[j * nslot:(j + 1) * nslot, :]
            kf = subf + jnp.float32(c * kc + j * nslot)
            b = dv < acc_val
            acc_val = jnp.where(b, dv, acc_val)
            acc_kf = jnp.where(b, kf, acc_kf)
    m = jnp.min(acc_val, axis=0, keepdims=True)
    cand = jnp.where(acc_val == m, acc_kf, jnp.float32(2.0**24))
    idxf = jnp.min(cand, axis=0, keepdims=True)
    lat_ref[...] = idxf.astype(jnp.int32).reshape(tq)


def _tc_argmin(z, cb, *, tq=2048, kc=2048, nslot=8, interpret=False):
    b, d, t = z.shape
    nq = b * t
    k = cb.shape[0]
    tpb = t // tq
    body = functools.partial(_argmin_body, kc=kc, nslot=nslot)
    return pl.pallas_call(
        body,
        grid=(nq // tq,),
        in_specs=[
            pl.BlockSpec((1, d, tq), lambda i: (i // tpb, 0, i % tpb)),
            pl.BlockSpec((k, d), lambda i: (0, 0)),
        ],
        out_specs=pl.BlockSpec((tq,), lambda i: (i,)),
        out_shape=jax.ShapeDtypeStruct((nq,), jnp.int32),
        interpret=interpret,
    )(z, cb)


def _sc_gather(cb, lat):
    k, d = cb.shape
    nq = lat.shape[0]
    info = plsc.get_sparse_core_info()
    nw = info.num_cores * info.num_subcores
    b_per_w = nq // nw
    chunk = min(b_per_w, 128)
    n_chunks = b_per_w // chunk
    mesh = plsc.VectorSubcoreMesh(core_axis_name="c", subcore_axis_name="s")

    @functools.partial(
        pl.kernel,
        mesh=mesh,
        compiler_params=pltpu.CompilerParams(use_tc_tiling_on_sc=False),
        out_type=jax.ShapeDtypeStruct((nq, d), jnp.float32),
        scratch_types=[
            pltpu.VMEM((b_per_w,), jnp.int32),
            pltpu.VMEM((chunk, d), jnp.float32),
            pltpu.SemaphoreType.DMA,
        ],
    )
    def gather_kernel(cb_hbm, idx_hbm, out_hbm, idx_v, rows_v, sem):
        wid = lax.axis_index("s") * info.num_cores + lax.axis_index("c")
        base = wid * b_per_w
        pltpu.sync_copy(idx_hbm.at[pl.ds(base, b_per_w)], idx_v)
        for j in range(n_chunks):
            pltpu.async_copy(cb_hbm.at[idx_v.at[pl.ds(j * chunk, chunk)]],
                             rows_v, sem).wait()
            pltpu.sync_copy(rows_v, out_hbm.at[pl.ds(base + j * chunk, chunk)])

    return gather_kernel(cb, lat)


def _finish_body(z_ref, q_ref, qst_ref, qt_ref):
    zb = z_ref[0]
    qb = q_ref[...]
    qt = qb.T
    qst_ref[0] = zb + (qt - zb)
    qt_ref[0] = qt


def _tc_finish(z, q, *, interpret=False):
    b, d, t = z.shape
    out = jax.ShapeDtypeStruct((b, d, t), jnp.float32)
    return pl.pallas_call(
        _finish_body,
        grid=(b,),
        in_specs=[
            pl.BlockSpec((1, d, t), lambda i: (i, 0, 0)),
            pl.BlockSpec((t, d), lambda i: (i, 0)),
        ],
        out_specs=[
            pl.BlockSpec((1, d, t), lambda i: (i, 0, 0)),
            pl.BlockSpec((1, d, t), lambda i: (i, 0, 0)),
        ],
        out_shape=[out, out],
        interpret=interpret,
    )(z, q)


def kernel(z, codebook):
    b, d, t = z.shape
    nq = b * t
    lat = _tc_argmin(z, codebook)
    q = _sc_gather(codebook, lat)
    qst_t, q_t = _tc_finish(z, q)
    return (qst_t, q_t)


# speedup vs baseline: 1.0965x; 1.0117x over previous
import functools

import jax
import jax.numpy as jnp
from jax import lax
from jax.experimental import pallas as pl
from jax.experimental.pallas import tpu as pltpu
from jax.experimental.pallas import tpu_sc as plsc


def _argmin_body(z_ref, cb_ref, lat_ref, *, kc, nslot):
    zb = z_ref[0]
    zs = zb + zb
    tq = zs.shape[1]
    x2 = jnp.sum(zb * zb, axis=0, keepdims=True)
    k_total = cb_ref.shape[0]
    nrow = kc // nslot
    subf = lax.broadcasted_iota(jnp.int32, (nslot, tq), 0).astype(jnp.float32)
    acc_val = jnp.full((nslot, tq), jnp.inf, jnp.float32)
    acc_kf = jnp.zeros((nslot, tq), jnp.float32)
    for c in range(k_total // kc):
        cbc = cb_ref[pl.ds(c * kc, kc), :]
        c2c = jnp.sum(cbc * cbc, axis=1, keepdims=True)
        dots2 = lax.dot_general(
            cbc, zs, (((1,), (0,)), ((), ())),
            preferred_element_type=jnp.float32)
        dist = (x2 - dots2) + c2c
        for j in range(nrow):
            dv = dist[j * nslot:(j + 1) * nslot, :]
            kf = subf + jnp.float32(c * kc + j * nslot)
            b = dv < acc_val
            acc_val = jnp.where(b, dv, acc_val)
            acc_kf = jnp.where(b, kf, acc_kf)
    m = jnp.min(acc_val, axis=0, keepdims=True)
    cand = jnp.where(acc_val == m, acc_kf, jnp.float32(2.0**24))
    idxf = jnp.min(cand, axis=0, keepdims=True)
    lat_ref[...] = idxf.astype(jnp.int32).reshape(tq)


def _tc_argmin(z, cb, *, tq=2048, kc=2048, nslot=8, interpret=False):
    b, d, t = z.shape
    nq = b * t
    k = cb.shape[0]
    tpb = t // tq
    body = functools.partial(_argmin_body, kc=kc, nslot=nslot)
    return pl.pallas_call(
        body,
        grid=(nq // tq,),
        in_specs=[
            pl.BlockSpec((1, d, tq), lambda i: (i // tpb, 0, i % tpb)),
            pl.BlockSpec((k, d), lambda i: (0, 0)),
        ],
        out_specs=pl.BlockSpec((tq,), lambda i: (i,)),
        out_shape=jax.ShapeDtypeStruct((nq,), jnp.int32),
        interpret=interpret,
    )(z, cb)


def _sc_gather(cb, lat):
    k, d = cb.shape
    nq = lat.shape[0]
    info = plsc.get_sparse_core_info()
    nw = info.num_cores * info.num_subcores
    b_per_w = nq // nw
    chunk = min(b_per_w, 128)
    n_chunks = b_per_w // chunk
    mesh = plsc.VectorSubcoreMesh(core_axis_name="c", subcore_axis_name="s")

    @functools.partial(
        pl.kernel,
        mesh=mesh,
        compiler_params=pltpu.CompilerParams(use_tc_tiling_on_sc=False),
        out_type=jax.ShapeDtypeStruct((nq, d), jnp.float32),
        scratch_types=[
            pltpu.VMEM((b_per_w,), jnp.int32),
            pltpu.VMEM((n_chunks, chunk, d), jnp.float32),
            pltpu.SemaphoreType.DMA,
        ],
    )
    def gather_kernel(cb_hbm, idx_hbm, out_hbm, idx_v, rows_v, sem):
        wid = lax.axis_index("s") * info.num_cores + lax.axis_index("c")
        base = wid * b_per_w
        pltpu.sync_copy(idx_hbm.at[pl.ds(base, b_per_w)], idx_v)
        copies = [
            pltpu.async_copy(cb_hbm.at[idx_v.at[pl.ds(j * chunk, chunk)]],
                             rows_v.at[j], sem)
            for j in range(n_chunks)
        ]
        for j in range(n_chunks):
            copies[j].wait()
            pltpu.sync_copy(rows_v.at[j],
                            out_hbm.at[pl.ds(base + j * chunk, chunk)])

    return gather_kernel(cb, lat)


def _finish_body(z_ref, q_ref, qst_ref, qt_ref):
    zb = z_ref[0]
    qb = q_ref[...]
    qt = qb.T
    qst_ref[0] = zb + (qt - zb)
    qt_ref[0] = qt


def _tc_finish(z, q, *, interpret=False):
    b, d, t = z.shape
    out = jax.ShapeDtypeStruct((b, d, t), jnp.float32)
    return pl.pallas_call(
        _finish_body,
        grid=(b,),
        in_specs=[
            pl.BlockSpec((1, d, t), lambda i: (i, 0, 0)),
            pl.BlockSpec((t, d), lambda i: (i, 0)),
        ],
        out_specs=[
            pl.BlockSpec((1, d, t), lambda i: (i, 0, 0)),
            pl.BlockSpec((1, d, t), lambda i: (i, 0, 0)),
        ],
        out_shape=[out, out],
        interpret=interpret,
    )(z, q)


def kernel(z, codebook):
    b, d, t = z.shape
    nq = b * t
    lat = _tc_argmin(z, codebook)
    q = _sc_gather(codebook, lat)
    qst_t, q_t = _tc_finish(z, q)
    return (qst_t, q_t)
